# Initial kernel scaffold; baseline (speedup 1.0000x reference)
import jax
import jax.numpy as jnp
from jax.experimental import pallas as pl


def kernel(hidden, edges, n_node, kgemb, left_num, rela_embed, Ws, Wr, Wkg_W, Wkg_b, walpha_W, walpha_b, Wh):
    raise NotImplementedError("write your pallas kernel here")


# speedup vs baseline: 2.9521x; 2.9521x over previous
import functools

import jax
import jax.numpy as jnp
from jax import lax
from jax.experimental import pallas as pl
from jax.experimental.pallas import tpu as pltpu
from jax.experimental.pallas import tpu_sc as plsc

N_NODE = 10000
D = 128
L = 16
NC, NS = 2, 16
NW = NC * NS
EW = 10080
E_PAD = NW * EW
SUPER = 1008
NSUP = EW // SUPER
CH = 48
NCH = SUPER // CH
GR = CH // L
N_PAD = 10240
RW = N_PAD // NS
ZR = 32
NZ = RW // ZR
PAD_ROWS = 10048


def _tables_body(hid_ref, rel_ref, ws_ref, wr_ref, node_ref, relo_ref):
    h = hid_ref[...]
    r = rel_ref[...]
    node_ref[:, :D] = h
    node_ref[:, D:] = lax.dot_general(
        h, ws_ref[...], (((1,), (1,)), ((), ())),
        preferred_element_type=jnp.float32)
    relo_ref[:, :D] = r
    relo_ref[:, D:] = lax.dot_general(
        r, wr_ref[...], (((1,), (1,)), ((), ())),
        preferred_element_type=jnp.float32)


def _build_tables(hid_p, rel_p, ws, wr):
    nblk = 8
    rows = PAD_ROWS // nblk
    return pl.pallas_call(
        _tables_body,
        grid=(nblk,),
        in_specs=[
            pl.BlockSpec((rows, D), lambda i: (i, 0)),
            pl.BlockSpec((rows, D), lambda i: (i, 0)),
            pl.BlockSpec((D, D), lambda i: (0, 0)),
            pl.BlockSpec((D, D), lambda i: (0, 0)),
        ],
        out_specs=[
            pl.BlockSpec((rows, 2 * D), lambda i: (i, 0)),
            pl.BlockSpec((rows, 2 * D), lambda i: (i, 0)),
        ],
        out_shape=[
            jax.ShapeDtypeStruct((PAD_ROWS, 2 * D), jnp.float32),
            jax.ShapeDtypeStruct((PAD_ROWS, 2 * D), jnp.float32),
        ],
    )(hid_p, rel_p, ws, wr)


def _c4_body(kg_ref, wkg_ref, wb_ref, out_ref):
    kg = kg_ref[...]
    w1 = wkg_ref[:, :D]
    w2 = wkg_ref[:, D:]
    kg1 = lax.dot_general(kg, w1, (((1,), (1,)), ((), ())),
                          preferred_element_type=jnp.float32)
    kg2 = lax.dot_general(kg, w2, (((1,), (1,)), ((), ())),
                          preferred_element_type=jnp.float32)
    c = kg1[:, None, :] + kg2[None, :, :] + wb_ref[...][None, None, :]
    c = c.reshape(4, D)
    out_ref[...] = jnp.concatenate([c, jnp.zeros((4, D), jnp.float32)], axis=0)


def _build_c4(kgemb, wkg_w, wkg_b):
    return pl.pallas_call(
        _c4_body,
        out_shape=jax.ShapeDtypeStruct((8, D), jnp.float32),
    )(kgemb, wkg_w, wkg_b)


def _lane_sum(v):
    lanes = lax.iota(jnp.int32, L)
    dnums = lax.GatherDimensionNumbers(
        offset_dims=(), collapsed_slice_dims=(0,), start_index_map=(0,))
    for sh in (1, 2, 4, 8):
        perm = lax.bitwise_xor(lanes, jnp.full((L,), sh, jnp.int32))
        v = v + lax.gather(v, perm[:, None], dnums, slice_sizes=(1,),
                           mode=lax.GatherScatterMode.PROMISE_IN_BOUNDS)
    return v


def _sc_body(node_hbm, rel_hbm, c4_hbm, w_hbm, b0_hbm, left_hbm, ehd_hbm,
             erl_hbm, etl_hbm, esb_hbm, eob_hbm,
             out_hbm, acc_sh, nrows, rrows, msg, zbuf, mhd, mrl, mtl, msb,
             mob, subi, reli, obji, c4v, wv, b0v, leftv, sem0, sem1):
    cid = lax.axis_index("c")
    sid = lax.axis_index("s")
    wid = cid * NS + sid

    pltpu.sync_copy(c4_hbm, c4v)
    pltpu.sync_copy(w_hbm, wv)
    pltpu.sync_copy(b0_hbm, b0v)
    pltpu.sync_copy(left_hbm, leftv)

    zero = jnp.zeros((L,), jnp.float32)

    def _zrow(r, carry):
        for k in range(8):
            zbuf[r, pl.ds(k * L, L)] = zero
        return carry

    lax.fori_loop(0, ZR, _zrow, 0)
    for i in range(NZ):
        pltpu.sync_copy(zbuf, acc_sh.at[pl.ds(sid * RW + i * ZR, ZR)])
    plsc.subcore_barrier()

    wk = tuple(wv[pl.ds(k * L, L)] for k in range(8))
    b0 = b0v[...]
    lft = leftv[...]
    twos = jnp.full((L,), 2, jnp.int32)
    zeros_i = jnp.zeros((L,), jnp.int32)
    kconst = tuple(lax.iota(jnp.int32, L) + jnp.full((L,), k * L, jnp.int32)
                   for k in range(8))

    ebase = wid * EW

    def _super(s_i, carry):
        sb = ebase + s_i * SUPER
        for col, buf in ((ehd_hbm, mhd), (erl_hbm, mrl), (etl_hbm, mtl),
                         (esb_hbm, msb), (eob_hbm, mob)):
            pltpu.sync_copy(col.at[pl.ds(sb, SUPER)], buf)

        def _chunk(c, carry2):
            base = c * CH
            for g in range(GR):
                subi[0, 0, pl.ds(g * L, L)] = msb[pl.ds(base + g * L, L)]
                reli[0, 0, pl.ds(g * L, L)] = mrl[pl.ds(base + g * L, L)]
                obji[0, 0, pl.ds(g * L, L)] = mob[pl.ds(base + g * L, L)]
            cp0 = pltpu.async_copy(node_hbm.at[subi.at[0, 0]], nrows, sem0)
            cp1 = pltpu.async_copy(rel_hbm.at[reli.at[0, 0]], rrows, sem1)
            cp0.wait()
            cp1.wait()

            for g in range(GR):
                hd = mhd[pl.ds(base + g * L, L)]
                tl = mtl[pl.ds(base + g * L, L)]
                kv = (jnp.where(hd >= lft, twos, zeros_i)
                      + (tl >= lft).astype(jnp.int32))
                for j2 in range(L):
                    j = g * L + j2
                    kgs = lax.broadcast(kv[j2] * D, (L,))
                    acc = jnp.zeros((L,), jnp.float32)
                    for k in range(8):
                        a = nrows[j, pl.ds(D + k * L, L)]
                        b = rrows[j, pl.ds(D + k * L, L)]
                        cc = plsc.load_gather(c4v, [kgs + kconst[k]])
                        acc = acc + jnp.maximum(a + b + cc, 0.0) * wk[k]
                    sv = _lane_sum(acc) + b0
                    av = 1.0 / (1.0 + jnp.exp(-sv))
                    for k in range(8):
                        m = (nrows[j, pl.ds(k * L, L)]
                             + rrows[j, pl.ds(k * L, L)]) * av
                        msg[j, pl.ds(k * L, L)] = m

            pltpu.sync_copy(msg, acc_sh.at[obji.at[0, 0]], add=True)
            return carry2

        lax.fori_loop(0, NCH, _chunk, 0)
        return carry

    lax.fori_loop(0, NSUP, _super, 0)
    plsc.subcore_barrier()

    for i in range(NZ):
        r0 = sid * RW + i * ZR
        pltpu.sync_copy(acc_sh.at[pl.ds(r0, ZR)], zbuf)
        pltpu.sync_copy(zbuf, out_hbm.at[cid, pl.ds(r0, ZR)])


def _sc_edges(node_tab, rel_tab, c4, wvec, b0vec, leftvec, ecols):
    ehd, erl, etl, esb, eob = ecols
    mesh = plsc.VectorSubcoreMesh(core_axis_name="c", subcore_axis_name="s",
                                  num_cores=NC, num_subcores=NS)
    fn = pl.kernel(
        _sc_body,
        out_type=jax.ShapeDtypeStruct((NC, N_PAD, D), jnp.float32),
        mesh=mesh,
        compiler_params=pltpu.CompilerParams(needs_layout_passes=False),
        scratch_types=[
            pltpu.VMEM_SHARED((N_PAD, D), jnp.float32),
            pltpu.VMEM((CH, 2 * D), jnp.float32),
            pltpu.VMEM((CH, 2 * D), jnp.float32),
            pltpu.VMEM((CH, D), jnp.float32),
            pltpu.VMEM((ZR, D), jnp.float32),
            pltpu.VMEM((SUPER,), jnp.int32),
            pltpu.VMEM((SUPER,), jnp.int32),
            pltpu.VMEM((SUPER,), jnp.int32),
            pltpu.VMEM((SUPER,), jnp.int32),
            pltpu.VMEM((SUPER,), jnp.int32),
            pltpu.VMEM((1, 1, CH), jnp.int32),
            pltpu.VMEM((1, 1, CH), jnp.int32),
            pltpu.VMEM((1, 1, CH), jnp.int32),
            pltpu.VMEM((4 * D,), jnp.float32),
            pltpu.VMEM((D,), jnp.float32),
            pltpu.VMEM((L,), jnp.float32),
            pltpu.VMEM((L,), jnp.int32),
            pltpu.SemaphoreType.DMA,
            pltpu.SemaphoreType.DMA,
        ],
    )
    return fn(node_tab, rel_tab, c4, wvec, b0vec, leftvec,
              ehd, erl, etl, esb, eob)


def _fin_body(p_ref, wh_ref, out_ref):
    s = p_ref[0] + p_ref[1]
    out_ref[...] = lax.dot_general(
        s, wh_ref[...], (((1,), (1,)), ((), ())),
        preferred_element_type=jnp.float32)


def _finish(partials, wh):
    nblk = 8
    rows = N_PAD // nblk
    return pl.pallas_call(
        _fin_body,
        grid=(nblk,),
        in_specs=[
            pl.BlockSpec((NC, rows, D), lambda i: (0, i, 0)),
            pl.BlockSpec((D, D), lambda i: (0, 0)),
        ],
        out_specs=pl.BlockSpec((rows, D), lambda i: (i, 0)),
        out_shape=jax.ShapeDtypeStruct((N_PAD, D), jnp.float32),
    )(partials, wh)


def kernel(hidden, edges, n_node, kgemb, left_num, rela_embed, Ws, Wr,
           Wkg_W, Wkg_b, walpha_W, walpha_b, Wh):
    hid_p = jnp.pad(hidden, ((0, PAD_ROWS - hidden.shape[0]), (0, 0)))
    rel_p = jnp.pad(rela_embed, ((0, PAD_ROWS - rela_embed.shape[0]), (0, 0)))
    e32 = edges.astype(jnp.int32)
    npad = E_PAD - e32.shape[0]
    objc = jnp.minimum(e32[:, 5], hidden.shape[0] - 1)
    objc = jnp.pad(objc, (0, npad), constant_values=N_NODE)
    ecols = tuple(jnp.pad(e32[:, c], (0, npad)) for c in (1, 2, 3, 4)) + (objc,)
    wvec = walpha_W.reshape(D)
    b0vec = jnp.broadcast_to(walpha_b.reshape(1), (L,)).astype(jnp.float32)
    leftvec = jnp.full((L,), left_num, jnp.int32)

    node_tab, rel_tab = _build_tables(hid_p, rel_p, Ws, Wr)
    c4 = _build_c4(kgemb, Wkg_W, Wkg_b)[:4].reshape(4 * D)
    partials = _sc_edges(node_tab, rel_tab, c4, wvec, b0vec, leftvec, ecols)
    return _finish(partials, Wh)[:N_NODE]


# speedup vs baseline: 4.0711x; 1.3791x over previous
import functools

import jax
import jax.numpy as jnp
from jax import lax
from jax.experimental import pallas as pl
from jax.experimental.pallas import tpu as pltpu
from jax.experimental.pallas import tpu_sc as plsc

N_NODE = 10000
D = 128
L = 16
NC, NS = 2, 16
NW = NC * NS
EW = 10240
E_PAD = NW * EW
SUPER = 1280
NSUP = EW // SUPER
CH = 32
NCH = SUPER // CH
NP = NCH // 2
GR = CH // L
N_PAD = 10240
RW = N_PAD // NS
ZR = 16
NZ = RW // ZR
PAD_ROWS = 10048


def _tables_body(hid_ref, rel_ref, ws_ref, wr_ref, node_ref, relo_ref):
    h = hid_ref[...]
    r = rel_ref[...]
    node_ref[:, :D] = h
    node_ref[:, D:] = lax.dot_general(
        h, ws_ref[...], (((1,), (1,)), ((), ())),
        preferred_element_type=jnp.float32)
    relo_ref[:, :D] = r
    relo_ref[:, D:] = lax.dot_general(
        r, wr_ref[...], (((1,), (1,)), ((), ())),
        preferred_element_type=jnp.float32)


def _build_tables(hid_p, rel_p, ws, wr):
    nblk = 8
    rows = PAD_ROWS // nblk
    return pl.pallas_call(
        _tables_body,
        grid=(nblk,),
        in_specs=[
            pl.BlockSpec((rows, D), lambda i: (i, 0)),
            pl.BlockSpec((rows, D), lambda i: (i, 0)),
            pl.BlockSpec((D, D), lambda i: (0, 0)),
            pl.BlockSpec((D, D), lambda i: (0, 0)),
        ],
        out_specs=[
            pl.BlockSpec((rows, 2 * D), lambda i: (i, 0)),
            pl.BlockSpec((rows, 2 * D), lambda i: (i, 0)),
        ],
        out_shape=[
            jax.ShapeDtypeStruct((PAD_ROWS, 2 * D), jnp.float32),
            jax.ShapeDtypeStruct((PAD_ROWS, 2 * D), jnp.float32),
        ],
    )(hid_p, rel_p, ws, wr)


def _c4_body(kg_ref, wkg_ref, wb_ref, out_ref):
    kg = kg_ref[...]
    w1 = wkg_ref[:, :D]
    w2 = wkg_ref[:, D:]
    kg1 = lax.dot_general(kg, w1, (((1,), (1,)), ((), ())),
                          preferred_element_type=jnp.float32)
    kg2 = lax.dot_general(kg, w2, (((1,), (1,)), ((), ())),
                          preferred_element_type=jnp.float32)
    c = kg1[:, None, :] + kg2[None, :, :] + wb_ref[...][None, None, :]
    c = c.reshape(4, D)
    out_ref[...] = jnp.concatenate([c, jnp.zeros((4, D), jnp.float32)], axis=0)


def _build_c4(kgemb, wkg_w, wkg_b):
    return pl.pallas_call(
        _c4_body,
        out_shape=jax.ShapeDtypeStruct((8, D), jnp.float32),
    )(kgemb, wkg_w, wkg_b)


def _lane_sum(v):
    lanes = lax.iota(jnp.int32, L)
    dnums = lax.GatherDimensionNumbers(
        offset_dims=(), collapsed_slice_dims=(0,), start_index_map=(0,))
    for sh in (1, 2, 4, 8):
        perm = lax.bitwise_xor(lanes, jnp.full((L,), sh, jnp.int32))
        v = v + lax.gather(v, perm[:, None], dnums, slice_sizes=(1,),
                           mode=lax.GatherScatterMode.PROMISE_IN_BOUNDS)
    return v


def _sc_body(node_hbm, rel_hbm, c4_hbm, w_hbm, b0_hbm, ekg_hbm,
             erl_hbm, esb_hbm, eob_hbm,
             out_hbm, acc_sh, nrows0, nrows1, rrows0, rrows1, msg, zbuf,
             mkg, mrl, msb, mob, subi0, subi1, reli0, reli1, obji0, obji1,
             c4v, wv, b0v, semn0, semr0, semn1, semr1):
    cid = lax.axis_index("c")
    sid = lax.axis_index("s")
    wid = cid * NS + sid

    pltpu.sync_copy(c4_hbm, c4v)
    pltpu.sync_copy(w_hbm, wv)
    pltpu.sync_copy(b0_hbm, b0v)

    zero = jnp.zeros((L,), jnp.float32)

    def _zrow(r, carry):
        for k in range(8):
            zbuf[r, pl.ds(k * L, L)] = zero
        return carry

    lax.fori_loop(0, ZR, _zrow, 0)
    for i in range(NZ):
        pltpu.sync_copy(zbuf, acc_sh.at[pl.ds(sid * RW + i * ZR, ZR)])
    plsc.subcore_barrier()

    wk = tuple(wv[pl.ds(k * L, L)] for k in range(8))
    b0 = b0v[...]
    kconst = tuple(lax.iota(jnp.int32, L) + jnp.full((L,), k * L, jnp.int32)
                   for k in range(8))

    ebase = wid * EW
    slots = ((nrows0, rrows0, subi0, reli0, obji0, semn0, semr0),
             (nrows1, rrows1, subi1, reli1, obji1, semn1, semr1))

    def _prep(c, slot):
        _, _, su, re_, ob, _, _ = slots[slot]
        for g in range(GR):
            su[0, 0, pl.ds(g * L, L)] = msb[pl.ds(c * CH + g * L, L)]
            re_[0, 0, pl.ds(g * L, L)] = mrl[pl.ds(c * CH + g * L, L)]
            ob[0, 0, pl.ds(g * L, L)] = mob[pl.ds(c * CH + g * L, L)]

    def _fire(slot):
        nr, rr, su, re_, _, sn, sr = slots[slot]
        pltpu.async_copy(node_hbm.at[su.at[0, 0]], nr, sn)
        pltpu.async_copy(rel_hbm.at[re_.at[0, 0]], rr, sr)

    def _wait(slot):
        nr, rr, su, re_, _, sn, sr = slots[slot]
        pltpu.make_async_copy(node_hbm.at[su.at[0, 0]], nr, sn).wait()
        pltpu.make_async_copy(rel_hbm.at[re_.at[0, 0]], rr, sr).wait()

    lane_dn = lax.GatherDimensionNumbers(
        offset_dims=(), collapsed_slice_dims=(0,), start_index_map=(0,))

    def _compute_scatter(c, slot):
        nr, rr, _, _, ob, _, _ = slots[slot]
        for g in range(GR):
            kv = mkg[pl.ds(c * CH + g * L, L)] * jnp.full((L,), D, jnp.int32)

            def _quad(q, carry):
                for u in range(4):
                    j2 = q * 4 + u
                    j = g * L + j2
                    perm = lax.broadcast(j2, (L,))
                    kgs = lax.gather(kv, perm[:, None], lane_dn,
                                     slice_sizes=(1,),
                                     mode=lax.GatherScatterMode.PROMISE_IN_BOUNDS)
                    acc = jnp.zeros((L,), jnp.float32)
                    for k in range(8):
                        a = nr[j, pl.ds(D + k * L, L)]
                        b = rr[j, pl.ds(D + k * L, L)]
                        cc = plsc.load_gather(c4v, [kgs + kconst[k]])
                        acc = acc + jnp.maximum(a + b + cc, 0.0) * wk[k]
                    sv = _lane_sum(acc) + b0
                    av = 1.0 / (1.0 + jnp.exp(-sv))
                    for k in range(8):
                        m = (nr[j, pl.ds(k * L, L)]
                             + rr[j, pl.ds(k * L, L)]) * av
                        msg[j, pl.ds(k * L, L)] = m
                return carry

            lax.fori_loop(0, 4, _quad, 0)
        pltpu.sync_copy(msg, acc_sh.at[ob.at[0, 0]], add=True)

    def _super(s_i, carry):
        sb = ebase + s_i * SUPER
        for col, buf in ((ekg_hbm, mkg), (erl_hbm, mrl), (esb_hbm, msb),
                         (eob_hbm, mob)):
            pltpu.sync_copy(col.at[pl.ds(sb, SUPER)], buf)

        _prep(0, 0)
        _fire(0)
        _prep(1, 1)
        _fire(1)

        def _pair(c2, carry2):
            c0 = c2 * 2
            _wait(0)
            _compute_scatter(c0, 0)
            _prep(c0 + 2, 0)
            _fire(0)
            _wait(1)
            _compute_scatter(c0 + 1, 1)
            _prep(c0 + 3, 1)
            _fire(1)
            return carry2

        lax.fori_loop(0, NP - 1, _pair, 0)
        _wait(0)
        _compute_scatter(NCH - 2, 0)
        _wait(1)
        _compute_scatter(NCH - 1, 1)
        return carry

    lax.fori_loop(0, NSUP, _super, 0)
    plsc.subcore_barrier()

    for i in range(NZ):
        r0 = sid * RW + i * ZR
        pltpu.sync_copy(acc_sh.at[pl.ds(r0, ZR)], zbuf)
        pltpu.sync_copy(zbuf, out_hbm.at[cid, pl.ds(r0, ZR)])


def _sc_edges(node_tab, rel_tab, c4, wvec, b0vec, ecols):
    ekg, erl, esb, eob = ecols
    mesh = plsc.VectorSubcoreMesh(core_axis_name="c", subcore_axis_name="s",
                                  num_cores=NC, num_subcores=NS)
    fn = pl.kernel(
        _sc_body,
        out_type=jax.ShapeDtypeStruct((NC, N_PAD, D), jnp.float32),
        mesh=mesh,
        compiler_params=pltpu.CompilerParams(needs_layout_passes=False),
        scratch_types=[
            pltpu.VMEM_SHARED((N_PAD, D), jnp.float32),
            pltpu.VMEM((CH, 2 * D), jnp.float32),
            pltpu.VMEM((CH, 2 * D), jnp.float32),
            pltpu.VMEM((CH, 2 * D), jnp.float32),
            pltpu.VMEM((CH, 2 * D), jnp.float32),
            pltpu.VMEM((CH, D), jnp.float32),
            pltpu.VMEM((ZR, D), jnp.float32),
            pltpu.VMEM((SUPER,), jnp.int32),
            pltpu.VMEM((SUPER,), jnp.int32),
            pltpu.VMEM((SUPER,), jnp.int32),
            pltpu.VMEM((SUPER,), jnp.int32),
            pltpu.VMEM((1, 1, CH), jnp.int32),
            pltpu.VMEM((1, 1, CH), jnp.int32),
            pltpu.VMEM((1, 1, CH), jnp.int32),
            pltpu.VMEM((1, 1, CH), jnp.int32),
            pltpu.VMEM((1, 1, CH), jnp.int32),
            pltpu.VMEM((1, 1, CH), jnp.int32),
            pltpu.VMEM((4 * D,), jnp.float32),
            pltpu.VMEM((D,), jnp.float32),
            pltpu.VMEM((L,), jnp.float32),
            pltpu.SemaphoreType.DMA,
            pltpu.SemaphoreType.DMA,
            pltpu.SemaphoreType.DMA,
            pltpu.SemaphoreType.DMA,
        ],
    )
    return fn(node_tab, rel_tab, c4, wvec, b0vec, ekg, erl, esb, eob)


def _fin_body(p_ref, wh_ref, out_ref):
    s = p_ref[0] + p_ref[1]
    out_ref[...] = lax.dot_general(
        s, wh_ref[...], (((1,), (1,)), ((), ())),
        preferred_element_type=jnp.float32)


def _finish(partials, wh):
    nblk = 8
    rows = N_PAD // nblk
    return pl.pallas_call(
        _fin_body,
        grid=(nblk,),
        in_specs=[
            pl.BlockSpec((NC, rows, D), lambda i: (0, i, 0)),
            pl.BlockSpec((D, D), lambda i: (0, 0)),
        ],
        out_specs=pl.BlockSpec((rows, D), lambda i: (i, 0)),
        out_shape=jax.ShapeDtypeStruct((N_PAD, D), jnp.float32),
    )(partials, wh)


def kernel(hidden, edges, n_node, kgemb, left_num, rela_embed, Ws, Wr,
           Wkg_W, Wkg_b, walpha_W, walpha_b, Wh):
    hid_p = jnp.pad(hidden, ((0, PAD_ROWS - hidden.shape[0]), (0, 0)))
    rel_p = jnp.pad(rela_embed, ((0, PAD_ROWS - rela_embed.shape[0]), (0, 0)))
    e32 = edges.astype(jnp.int32)
    npad = E_PAD - e32.shape[0]
    objc = jnp.minimum(e32[:, 5], hidden.shape[0] - 1)
    objc = jnp.pad(objc, (0, npad), constant_values=N_NODE)
    kidx = 2 * (e32[:, 1] >= left_num).astype(jnp.int32) \
        + (e32[:, 3] >= left_num).astype(jnp.int32)
    ecols = (jnp.pad(kidx, (0, npad)), jnp.pad(e32[:, 2], (0, npad)),
             jnp.pad(e32[:, 4], (0, npad)), objc)
    wvec = walpha_W.reshape(D)
    b0vec = jnp.broadcast_to(walpha_b.reshape(1), (L,)).astype(jnp.float32)

    node_tab, rel_tab = _build_tables(hid_p, rel_p, Ws, Wr)
    c4 = _build_c4(kgemb, Wkg_W, Wkg_b)[:4].reshape(4 * D)
    partials = _sc_edges(node_tab, rel_tab, c4, wvec, b0vec, ecols)
    return _finish(partials, Wh)[:N_NODE]
